# baseline (speedup 1.0000x reference)
import functools
import math

import jax
import jax.numpy as jnp
from jax import lax
from jax.experimental import pallas as pl
from jax.experimental.pallas import tpu as pltpu
from jax.experimental.pallas import tpu_sc as plsc

D_MODEL = 64
SCALE = 1.0 / math.sqrt(D_MODEL)


def kernel(x, lut):
    B = x.shape[0] * x.shape[1]
    V, D = lut.shape
    flat_idx = x.reshape(B).astype(jnp.int32)
    out = _call(flat_idx, lut, B, V, D)
    return out.reshape(x.shape[0], x.shape[1], D)


@functools.partial(jax.jit, static_argnums=(2, 3, 4))
def _call(flat_idx, lut, B, V, D):
    info = plsc.get_sparse_core_info()
    NC, NS = info.num_cores, info.num_subcores
    NW = NC * NS
    b_per_w = B // NW
    C = 1024
    n_chunks = b_per_w // C
    mesh = plsc.VectorSubcoreMesh(core_axis_name="c", subcore_axis_name="s")

    def body(idx_hbm, table_hbm, out_hbm, idx_v, rows_v, sem):
        wid = lax.axis_index("s") * NC + lax.axis_index("c")
        base = wid * b_per_w

        def chunk_body(ci, carry):
            off = base + ci * C
            pltpu.sync_copy(idx_hbm.at[pl.ds(off, C)], idx_v)
            pltpu.async_copy(table_hbm.at[idx_v], rows_v, sem).wait()

            def scale_row(r, c2):
                for j in range(D // 16):
                    sl = (r, pl.ds(j * 16, 16))
                    rows_v[sl] = rows_v[sl] * SCALE
                return c2

            lax.fori_loop(0, C, scale_row, 0)
            pltpu.sync_copy(rows_v, out_hbm.at[pl.ds(off, C)])
            return carry

        lax.fori_loop(0, n_chunks, chunk_body, 0)

    return pl.kernel(
        body,
        mesh=mesh,
        compiler_params=pltpu.CompilerParams(use_tc_tiling_on_sc=False),
        out_type=jax.ShapeDtypeStruct((B, D), jnp.float32),
        scratch_types=[
            pltpu.VMEM((C,), jnp.int32),
            pltpu.VMEM((C, D), jnp.float32),
            pltpu.SemaphoreType.DMA,
        ],
    )(flat_idx, lut)


# speedup vs baseline: 1.1030x; 1.1030x over previous
import functools
import math

import jax
import jax.numpy as jnp
from jax import lax
from jax.experimental import pallas as pl
from jax.experimental.pallas import tpu as pltpu
from jax.experimental.pallas import tpu_sc as plsc

D_MODEL = 64
SCALE = 1.0 / math.sqrt(D_MODEL)

C = 800
NBUF = 2
UNROLL = 8


def kernel(x, lut):
    B = x.shape[0] * x.shape[1]
    V, D = lut.shape
    flat_idx = x.reshape(B).astype(jnp.int32)
    out = _call(flat_idx, lut, B, V, D)
    return out.reshape(x.shape[0], x.shape[1], D)


@functools.partial(jax.jit, static_argnums=(2, 3, 4))
def _call(flat_idx, lut, B, V, D):
    info = plsc.get_sparse_core_info()
    NC, NS = info.num_cores, info.num_subcores
    NW = NC * NS
    b_per_w = B // NW
    n_chunks = b_per_w // C
    assert b_per_w % C == 0 and n_chunks % NBUF == 0
    n_super = n_chunks // NBUF
    mesh = plsc.VectorSubcoreMesh(core_axis_name="c", subcore_axis_name="s")

    def body(idx_hbm, table_hbm, out_hbm, idx_v, rows_v, gsem, wsem):
        wid = lax.axis_index("s") * NC + lax.axis_index("c")
        base = wid * b_per_w

        def scale_chunk(b):
            def scale_rows(i, c2):
                r0 = i * UNROLL
                for u in range(UNROLL):
                    for j in range(D // 16):
                        sl = (r0 + u, pl.ds(j * 16, 16))
                        rows_v[b][sl] = rows_v[b][sl] * SCALE
                return c2

            lax.fori_loop(0, C // UNROLL, scale_rows, 0)

        for b in range(NBUF):
            pltpu.sync_copy(idx_hbm.at[pl.ds(base + b * C, C)], idx_v[b])
            pltpu.async_copy(table_hbm.at[idx_v[b]], rows_v[b], gsem[b])

        def super_body(g, carry):
            for b in range(NBUF):
                off = base + (g * NBUF + b) * C
                nxt = off + NBUF * C
                pltpu.make_async_copy(table_hbm.at[idx_v[b]], rows_v[b],
                                      gsem[b]).wait()
                pltpu.sync_copy(idx_hbm.at[pl.ds(nxt, C)], idx_v[b])
                scale_chunk(b)
                pltpu.async_copy(rows_v[b], out_hbm.at[pl.ds(off, C)], wsem[b])
                pltpu.make_async_copy(rows_v[b], out_hbm.at[pl.ds(off, C)],
                                      wsem[b]).wait()
                pltpu.async_copy(table_hbm.at[idx_v[b]], rows_v[b], gsem[b])
            return carry

        lax.fori_loop(0, n_super - 1, super_body, 0)

        for b in range(NBUF):
            off = base + (n_chunks - NBUF + b) * C
            pltpu.make_async_copy(table_hbm.at[idx_v[b]], rows_v[b],
                                  gsem[b]).wait()
            scale_chunk(b)
            pltpu.async_copy(rows_v[b], out_hbm.at[pl.ds(off, C)], wsem[b])
        for b in range(NBUF):
            off = base + (n_chunks - NBUF + b) * C
            pltpu.make_async_copy(rows_v[b], out_hbm.at[pl.ds(off, C)],
                                  wsem[b]).wait()

    return pl.kernel(
        body,
        mesh=mesh,
        compiler_params=pltpu.CompilerParams(use_tc_tiling_on_sc=False),
        out_type=jax.ShapeDtypeStruct((B, D), jnp.float32),
        scratch_types=[
            [pltpu.VMEM((C,), jnp.int32) for _ in range(NBUF)],
            [pltpu.VMEM((C, D), jnp.float32) for _ in range(NBUF)],
            [pltpu.SemaphoreType.DMA for _ in range(NBUF)],
            [pltpu.SemaphoreType.DMA for _ in range(NBUF)],
        ],
    )(flat_idx, lut)
